# baseline (speedup 1.0000x reference)
import jax
import jax.numpy as jnp
from jax import lax
from jax.experimental import pallas as pl
from jax.experimental.pallas import tpu as pltpu
from jax.experimental.pallas import tpu_sc as plsc

_LANES = 128
_NSC = 2
_NSUB = 16
_NW = _NSC * _NSUB


def _cdiv(a, b):
    return (a + b - 1) // b


def _sc_mesh():
    return plsc.VectorSubcoreMesh(core_axis_name="c", subcore_axis_name="s")


def _tc_edge_prep_dst(edge_index, n, n_pad, ep):
    e = edge_index.shape[1]
    blk = 65536
    grid = ep // blk
    trash = n_pad - n

    def body(ei_ref, d_ref):
        gi = pl.program_id(0) * blk + lax.broadcasted_iota(jnp.int32, (blk,), 0)
        d_ref[...] = jnp.where(gi < e, ei_ref[1, :], n + gi % trash)

    return pl.pallas_call(
        body,
        grid=(grid,),
        in_specs=[pl.BlockSpec((2, blk), lambda b: (0, b))],
        out_specs=pl.BlockSpec((blk,), lambda b: (b,)),
        out_shape=jax.ShapeDtypeStruct((ep,), jnp.int32),
    )(edge_index)


def _tc_edge_prep_src(edge_index, n, ep):
    e = edge_index.shape[1]
    blk = 65536
    grid = ep // blk

    def body(ei_ref, s_ref, s2_ref):
        gi = pl.program_id(0) * blk + lax.broadcasted_iota(jnp.int32, (blk,), 0)
        s = jnp.where(gi < e, ei_ref[0, :], gi % n)
        s_ref[...] = s
        s2_ref[...] = 2 * s

    return pl.pallas_call(
        body,
        grid=(grid,),
        in_specs=[pl.BlockSpec((2, blk), lambda b: (0, b))],
        out_specs=[pl.BlockSpec((blk,), lambda b: (b,))] * 2,
        out_shape=[jax.ShapeDtypeStruct((ep,), jnp.int32)] * 2,
    )(edge_index)


def _sc_degree(dst2d, zeros16, ones16, n_pad, nch_w):
    rows_w = n_pad // _NSUB
    ngrp = rows_w // 16
    npair_w = rows_w // 2

    def body(dst_hbm, zeros_hbm, ones_hbm, outv_hbm, oute_hbm,
             didx_all, ones_v, cnt_v, deg_v, dege_v, acc, ssem):
        cid = lax.axis_index("c")
        sid = lax.axis_index("s")
        wid = cid * _NSUB + sid
        pltpu.sync_copy(zeros_hbm, acc.at[pl.ds(sid * rows_w, rows_w)])
        pltpu.sync_copy(ones_hbm, ones_v)
        pltpu.sync_copy(dst_hbm.at[pl.ds(wid * nch_w, nch_w)], didx_all)
        plsc.subcore_barrier()

        def fire(ci, _):
            pltpu.async_copy(ones_v, acc.at[didx_all.at[ci]], ssem, add=True)
            return ()

        def drain(ci, _):
            pltpu.make_async_copy(ones_v, acc.at[didx_all.at[ci]], ssem).wait()
            return ()

        lax.fori_loop(0, nch_w, fire, ())
        lax.fori_loop(0, nch_w, drain, ())
        plsc.subcore_barrier()

        pltpu.sync_copy(acc.at[pl.ds(sid * rows_w, rows_w)], cnt_v)
        riota = lax.iota(jnp.int32, 16)
        zidx = jnp.zeros((16,), jnp.int32)

        def compress(g, _):
            vals = plsc.load_gather(cnt_v, [g * 16 + riota, zidx])
            deg_v[pl.ds(g * 16, 16)] = vals
            return ()

        lax.fori_loop(0, ngrp, compress, ())
        pltpu.sync_copy(deg_v, outv_hbm.at[cid, pl.ds(sid * rows_w, rows_w)])

        def expand(r, _):
            v0 = cnt_v[2 * r, :]
            v1 = cnt_v[2 * r + 1, :]
            for k in range(4):
                dege_v[r, pl.ds(16 * k, 16)] = v0
            for k in range(4, 8):
                dege_v[r, pl.ds(16 * k, 16)] = v1
            return ()

        lax.fori_loop(0, npair_w, expand, ())
        pltpu.sync_copy(dege_v, oute_hbm.at[cid, pl.ds(sid * npair_w, npair_w)])

    fn = pl.kernel(
        body,
        out_type=[jax.ShapeDtypeStruct((_NSC, n_pad), jnp.float32),
                  jax.ShapeDtypeStruct((_NSC, n_pad // 2, 128), jnp.float32)],
        mesh=_sc_mesh(),
        compiler_params=pltpu.CompilerParams(use_tc_tiling_on_sc=False,
                                             needs_layout_passes=False),
        scratch_types=[
            pltpu.VMEM((nch_w, _LANES), jnp.int32),
            pltpu.VMEM((_LANES, 16), jnp.float32),
            pltpu.VMEM((rows_w, 16), jnp.float32),
            pltpu.VMEM((rows_w,), jnp.float32),
            pltpu.VMEM((npair_w, 128), jnp.float32),
            pltpu.VMEM_SHARED((n_pad, 16), jnp.float32),
            pltpu.SemaphoreType.DMA,
        ],
    )
    return fn(dst2d, zeros16, ones16)


def _sc_spmm(y, doubled_idx, src2d, dst2d, zeros_f, n, n_pad, nch_w):
    f = y.shape[1]
    rows_w = n_pad // _NSUB
    npairs = nch_w // 2
    nself = rows_w // _LANES
    full_tiles = n // rows_w
    rem = n % rows_w

    def body(y_hbm, src_hbm, dst_hbm, zeros_hbm, out_hbm,
             sidx_all, didx_all, sidx_self, didx_self, rows0, rows1, acc,
             gsem0, gsem1):
        cid = lax.axis_index("c")
        sid = lax.axis_index("s")
        wid = cid * _NSUB + sid

        if doubled_idx:
            pltpu.sync_copy(zeros_hbm, acc.at[pl.ds(sid * rows_w, rows_w)])
            riota = lax.iota(jnp.int32, 16)
            base_node = sid * rows_w
            for c in range(nself):
                for g in range(8):
                    nodes = base_node + (c * 128 + g * 16) + riota
                    didx_self[c, pl.ds(16 * g, 16)] = nodes
                    sidx_self[c, pl.ds(16 * g, 16)] = (
                        jnp.minimum(nodes, n - 1) * 2)
        else:
            @pl.when(jnp.logical_and(cid == 0, sid < full_tiles))
            def _():
                pltpu.sync_copy(y_hbm.at[pl.ds(sid * rows_w, rows_w)],
                                acc.at[pl.ds(sid * rows_w, rows_w)])

            @pl.when(jnp.logical_and(cid == 0, sid >= full_tiles))
            def _():
                if rem:
                    pltpu.sync_copy(y_hbm.at[pl.ds(sid * rows_w, rem)],
                                    acc.at[pl.ds(sid * rows_w, rem)])
                pltpu.sync_copy(
                    zeros_hbm.at[pl.ds(0, rows_w - rem)],
                    acc.at[pl.ds(sid * rows_w + rem, rows_w - rem)])

            @pl.when(cid != 0)
            def _():
                pltpu.sync_copy(zeros_hbm,
                                acc.at[pl.ds(sid * rows_w, rows_w)])

        pltpu.sync_copy(src_hbm.at[pl.ds(wid * nch_w, nch_w)], sidx_all)
        pltpu.sync_copy(dst_hbm.at[pl.ds(wid * nch_w, nch_w)], didx_all)
        plsc.subcore_barrier()

        pltpu.async_copy(y_hbm.at[sidx_all.at[0]], rows0, gsem0)

        def step(i, _):
            c0 = 2 * i
            c1 = c0 + 1
            pltpu.async_copy(y_hbm.at[sidx_all.at[c1]], rows1, gsem1)
            pltpu.make_async_copy(y_hbm.at[sidx_all.at[c0]], rows0, gsem0).wait()
            pltpu.sync_copy(rows0, acc.at[didx_all.at[c0]], add=True)
            cn = jnp.minimum(c0 + 2, nch_w - 1)
            pltpu.async_copy(y_hbm.at[sidx_all.at[cn]], rows0, gsem0)
            pltpu.make_async_copy(y_hbm.at[sidx_all.at[c1]], rows1, gsem1).wait()
            pltpu.sync_copy(rows1, acc.at[didx_all.at[c1]], add=True)
            return ()

        lax.fori_loop(0, npairs, step, ())
        pltpu.make_async_copy(y_hbm.at[sidx_all.at[nch_w - 1]], rows0,
                              gsem0).wait()
        if doubled_idx:
            for parity in range(2):
                lst = list(range(parity, nself, 2))

                @pl.when(cid == parity)
                def _(lst=lst):
                    bufs = ((rows0, gsem0), (rows1, gsem1))
                    for j in range(min(2, len(lst))):
                        pltpu.async_copy(y_hbm.at[sidx_self.at[lst[j]]],
                                         bufs[j][0], bufs[j][1])
                    for j, c in enumerate(lst):
                        buf, sem = bufs[j % 2]
                        pltpu.make_async_copy(y_hbm.at[sidx_self.at[c]],
                                              buf, sem).wait()
                        pltpu.sync_copy(buf, acc.at[didx_self.at[c]],
                                        add=True)
                        if j + 2 < len(lst):
                            pltpu.async_copy(
                                y_hbm.at[sidx_self.at[lst[j + 2]]], buf, sem)
        plsc.subcore_barrier()
        pltpu.sync_copy(acc.at[pl.ds(sid * rows_w, rows_w)],
                        out_hbm.at[cid, pl.ds(sid * rows_w, rows_w)])

    fn = pl.kernel(
        body,
        out_type=jax.ShapeDtypeStruct((_NSC, n_pad, f), jnp.float32),
        mesh=_sc_mesh(),
        compiler_params=pltpu.CompilerParams(use_tc_tiling_on_sc=False),
        scratch_types=[
            pltpu.VMEM((nch_w, _LANES), jnp.int32),
            pltpu.VMEM((nch_w, _LANES), jnp.int32),
            pltpu.VMEM((nself, _LANES), jnp.int32),
            pltpu.VMEM((nself, _LANES), jnp.int32),
            pltpu.VMEM((_LANES, f), jnp.float32),
            pltpu.VMEM((_LANES, f), jnp.float32),
            pltpu.VMEM_SHARED((n_pad, f), jnp.float32),
            pltpu.SemaphoreType.DMA,
            pltpu.SemaphoreType.DMA,
        ],
    )
    return fn(y, src2d, dst2d, zeros_f)


def _tc_scale_first(degv128, x, W, n):
    c_out = W.shape[1]
    xb = 1024
    grid = _cdiv(n, xb)

    def body(degv_ref, x_ref, w_ref, z_ref):
        pid = pl.program_id(0)
        nrow = xb // 128
        deg = (degv_ref[0, pl.ds(nrow * pid, nrow), :]
               + degv_ref[1, pl.ds(nrow * pid, nrow), :]) + 1.0
        dis_t = lax.transpose(lax.rsqrt(deg), (1, 0))
        dcol = jnp.concatenate(
            [dis_t[:, k:k + 1] for k in range(nrow)], axis=0)
        xw = jnp.dot(x_ref[...], w_ref[...],
                     preferred_element_type=jnp.float32)
        z_ref[...] = jnp.concatenate(
            [xw * dcol, jnp.zeros((xb, 128 - c_out), jnp.float32)], axis=1)

    return pl.pallas_call(
        body,
        grid=(grid,),
        in_specs=[
            pl.BlockSpec(degv128.shape, lambda b: (0, 0, 0)),
            pl.BlockSpec((xb, x.shape[1]), lambda b: (b, 0)),
            pl.BlockSpec((x.shape[1], c_out), lambda b: (0, 0)),
        ],
        out_specs=pl.BlockSpec((xb, 128), lambda b: (b, 0)),
        out_shape=jax.ShapeDtypeStruct((n, 128), jnp.float32),
    )(degv128, x, W)


def _tc_mid(up128, degE, n2):
    def body(up_ref, de_ref, v_ref):
        u = up_ref[0, :n2, :] + up_ref[1, :n2, :]
        deg = de_ref[0, :n2, :] + de_ref[1, :n2, :] + 1.0
        v_ref[...] = u / deg

    return pl.pallas_call(
        body,
        out_shape=jax.ShapeDtypeStruct((n2, 128), jnp.float32),
    )(up128, degE)


def _tc_final(wp128, degE, b2, P1, P2, n):
    c_out = b2.shape[1] // 2
    ob = P1.shape[0]
    pb = ob // 2
    grid = _cdiv(n, ob)

    def body(wp_ref, de_ref, b_ref, p1_ref, p2_ref, o_ref):
        w = wp_ref[0] + wp_ref[1]
        deg = de_ref[0] + de_ref[1] + 1.0
        logits = w * lax.rsqrt(deg) + b_ref[...]

        def lsm(l):
            m = jnp.max(l, axis=-1, keepdims=True)
            ex = jnp.exp(l - m)
            return l - (jnp.log(jnp.sum(ex, axis=-1, keepdims=True)) + m)

        o_ref[...] = (
            jnp.dot(p1_ref[...], lsm(logits[:, :c_out]),
                    preferred_element_type=jnp.float32,
                    precision=lax.Precision.HIGHEST)
            + jnp.dot(p2_ref[...], lsm(logits[:, c_out:]),
                      preferred_element_type=jnp.float32,
                      precision=lax.Precision.HIGHEST))

    return pl.pallas_call(
        body,
        grid=(grid,),
        in_specs=[
            pl.BlockSpec((_NSC, pb, 2 * c_out), lambda b: (0, b, 0)),
            pl.BlockSpec((_NSC, pb, 2 * c_out), lambda b: (0, b, 0)),
            pl.BlockSpec(b2.shape, lambda b: (0, 0)),
            pl.BlockSpec(P1.shape, lambda b: (0, 0)),
            pl.BlockSpec(P2.shape, lambda b: (0, 0)),
        ],
        out_specs=pl.BlockSpec((ob, c_out), lambda b: (b, 0)),
        out_shape=jax.ShapeDtypeStruct((n, c_out), jnp.float32),
    )(wp128, degE, b2, P1, P2)


def kernel(x, edge_index, W, b):
    n = x.shape[0]
    c_out = W.shape[1]
    e = edge_index.shape[1]
    n2 = n // 2

    n_pad = _cdiv(n + 1, 8 * _LANES) * 8 * _LANES
    nch = _cdiv(e, _LANES * _NW * 8) * _NW * 8
    nch_w = nch // _NW
    ep = nch * _LANES

    dst_f = _tc_edge_prep_dst(edge_index, n, n_pad, ep)
    src_f, src2_f = _tc_edge_prep_src(edge_index, n, ep)
    src_p = src_f.reshape(nch, _LANES)
    src2_p = src2_f.reshape(nch, _LANES)
    dst_p = dst_f.reshape(nch, _LANES)

    rows_w = n_pad // _NSUB
    zeros16 = jnp.zeros((rows_w, 16), jnp.float32)
    zerosf = jnp.zeros((rows_w, c_out), jnp.float32)
    ones16 = jnp.ones((_LANES, 16), jnp.float32)
    b2 = jnp.concatenate([b, b]).reshape(1, 2 * c_out)
    ob = 1024
    ii = jnp.arange(ob, dtype=jnp.int32)[:, None]
    qq = jnp.arange(ob // 2, dtype=jnp.int32)[None, :]
    P1 = ((ii % 2 == 0) & (qq == ii // 2)).astype(jnp.float32)
    P2 = ((ii % 2 == 1) & (qq == ii // 2)).astype(jnp.float32)

    degv, degE = _sc_degree(dst_p, zeros16, ones16, n_pad, nch_w)
    degv128 = degv.reshape(_NSC, n_pad // _LANES, _LANES)
    zwide = _tc_scale_first(degv128, x, W, n)
    up = _sc_spmm(zwide.reshape(2 * n, c_out), True,
                  src2_p, dst_p, zerosf, n, n_pad, nch_w)
    v128 = _tc_mid(up.reshape(_NSC, n_pad // 2, 2 * c_out), degE, n2)
    wp = _sc_spmm(v128.reshape(n, c_out), False,
                  src_p, dst_p, zerosf, n, n_pad, nch_w)
    return _tc_final(wp.reshape(_NSC, n_pad // 2, 2 * c_out), degE, b2,
                     P1, P2, n)


# speedup vs baseline: 1.1730x; 1.1730x over previous
import jax
import jax.numpy as jnp
from jax import lax
from jax.experimental import pallas as pl
from jax.experimental.pallas import tpu as pltpu
from jax.experimental.pallas import tpu_sc as plsc

_LANES = 128
_NSC = 2
_NSUB = 16
_NW = _NSC * _NSUB


def _cdiv(a, b):
    return (a + b - 1) // b


def _sc_mesh():
    return plsc.VectorSubcoreMesh(core_axis_name="c", subcore_axis_name="s")


def _tc_edge_prep_dst(edge_index, n, n_pad, ep):
    e = edge_index.shape[1]
    blk = 65536
    grid = ep // blk
    trash = n_pad - n

    def body(ei_ref, d_ref):
        gi = pl.program_id(0) * blk + lax.broadcasted_iota(jnp.int32, (blk,), 0)
        d_ref[...] = jnp.where(gi < e, ei_ref[1, :], n + gi % trash)

    return pl.pallas_call(
        body,
        grid=(grid,),
        in_specs=[pl.BlockSpec((2, blk), lambda b: (0, b))],
        out_specs=pl.BlockSpec((blk,), lambda b: (b,)),
        out_shape=jax.ShapeDtypeStruct((ep,), jnp.int32),
    )(edge_index)


def _tc_edge_prep_src(edge_index, n, ep):
    e = edge_index.shape[1]
    blk = 65536
    grid = ep // blk

    def body(ei_ref, s_ref, s2_ref):
        gi = pl.program_id(0) * blk + lax.broadcasted_iota(jnp.int32, (blk,), 0)
        s = jnp.where(gi < e, ei_ref[0, :], gi % n)
        s_ref[...] = s
        s2_ref[...] = 2 * s

    return pl.pallas_call(
        body,
        grid=(grid,),
        in_specs=[pl.BlockSpec((2, blk), lambda b: (0, b))],
        out_specs=[pl.BlockSpec((blk,), lambda b: (b,))] * 2,
        out_shape=[jax.ShapeDtypeStruct((ep,), jnp.int32)] * 2,
    )(edge_index)


def _sc_degree(dst2d, zeros16, ones16, n_pad, nch_w):
    rows_w = n_pad // _NSUB
    ngrp = rows_w // 16
    npair_w = rows_w // 2

    def body(dst_hbm, zeros_hbm, ones_hbm, outv_hbm, oute_hbm,
             didx_all, ones_v, cnt_v, deg_v, dege_v, acc, ssem):
        cid = lax.axis_index("c")
        sid = lax.axis_index("s")
        wid = cid * _NSUB + sid
        pltpu.sync_copy(zeros_hbm, acc.at[pl.ds(sid * rows_w, rows_w)])
        pltpu.sync_copy(ones_hbm, ones_v)
        pltpu.sync_copy(dst_hbm.at[pl.ds(wid * nch_w, nch_w)], didx_all)
        plsc.subcore_barrier()

        def fire(ci, _):
            pltpu.async_copy(ones_v, acc.at[didx_all.at[ci]], ssem, add=True)
            return ()

        def drain(ci, _):
            pltpu.make_async_copy(ones_v, acc.at[didx_all.at[ci]], ssem).wait()
            return ()

        lax.fori_loop(0, nch_w, fire, ())
        lax.fori_loop(0, nch_w, drain, ())
        plsc.subcore_barrier()

        pltpu.sync_copy(acc.at[pl.ds(sid * rows_w, rows_w)], cnt_v)
        riota = lax.iota(jnp.int32, 16)
        zidx = jnp.zeros((16,), jnp.int32)

        def compress(g, _):
            vals = plsc.load_gather(cnt_v, [g * 16 + riota, zidx])
            deg_v[pl.ds(g * 16, 16)] = vals
            return ()

        lax.fori_loop(0, ngrp, compress, ())
        pltpu.sync_copy(deg_v, outv_hbm.at[cid, pl.ds(sid * rows_w, rows_w)])

        def expand(r, _):
            v0 = cnt_v[2 * r, :]
            v1 = cnt_v[2 * r + 1, :]
            for k in range(4):
                dege_v[r, pl.ds(16 * k, 16)] = v0
            for k in range(4, 8):
                dege_v[r, pl.ds(16 * k, 16)] = v1
            return ()

        lax.fori_loop(0, npair_w, expand, ())
        pltpu.sync_copy(dege_v, oute_hbm.at[cid, pl.ds(sid * npair_w, npair_w)])

    fn = pl.kernel(
        body,
        out_type=[jax.ShapeDtypeStruct((_NSC, n_pad), jnp.float32),
                  jax.ShapeDtypeStruct((_NSC, n_pad // 2, 128), jnp.float32)],
        mesh=_sc_mesh(),
        compiler_params=pltpu.CompilerParams(use_tc_tiling_on_sc=False,
                                             needs_layout_passes=False),
        scratch_types=[
            pltpu.VMEM((nch_w, _LANES), jnp.int32),
            pltpu.VMEM((_LANES, 16), jnp.float32),
            pltpu.VMEM((rows_w, 16), jnp.float32),
            pltpu.VMEM((rows_w,), jnp.float32),
            pltpu.VMEM((npair_w, 128), jnp.float32),
            pltpu.VMEM_SHARED((n_pad, 16), jnp.float32),
            pltpu.SemaphoreType.DMA,
        ],
    )
    return fn(dst2d, zeros16, ones16)


def _sc_spmm(y, doubled_idx, src2d, dst2d, zeros_f, n, n_pad, nch_w):
    f = y.shape[1]
    rows_w = n_pad // _NSUB
    npairs = nch_w // 2
    nself = rows_w // _LANES
    full_tiles = n // rows_w
    rem = n % rows_w

    def body(y_hbm, src_hbm, dst_hbm, zeros_hbm, out_hbm,
             sidx_all, didx_all, sidx_self, didx_self, rows0, rows1, acc,
             gsem0, gsem1):
        cid = lax.axis_index("c")
        sid = lax.axis_index("s")
        wid = cid * _NSUB + sid

        if doubled_idx:
            pltpu.sync_copy(zeros_hbm, acc.at[pl.ds(sid * rows_w, rows_w)])
            riota = lax.iota(jnp.int32, 16)
            base_node = sid * rows_w
            for c in range(nself):
                for g in range(8):
                    nodes = base_node + (c * 128 + g * 16) + riota
                    didx_self[c, pl.ds(16 * g, 16)] = nodes
                    sidx_self[c, pl.ds(16 * g, 16)] = (
                        jnp.minimum(nodes, n - 1) * 2)
        else:
            @pl.when(jnp.logical_and(cid == 0, sid < full_tiles))
            def _():
                pltpu.sync_copy(y_hbm.at[pl.ds(sid * rows_w, rows_w)],
                                acc.at[pl.ds(sid * rows_w, rows_w)])

            @pl.when(jnp.logical_and(cid == 0, sid >= full_tiles))
            def _():
                if rem:
                    pltpu.sync_copy(y_hbm.at[pl.ds(sid * rows_w, rem)],
                                    acc.at[pl.ds(sid * rows_w, rem)])
                pltpu.sync_copy(
                    zeros_hbm.at[pl.ds(0, rows_w - rem)],
                    acc.at[pl.ds(sid * rows_w + rem, rows_w - rem)])

            @pl.when(cid != 0)
            def _():
                pltpu.sync_copy(zeros_hbm,
                                acc.at[pl.ds(sid * rows_w, rows_w)])

        pltpu.sync_copy(src_hbm.at[pl.ds(wid * nch_w, nch_w)], sidx_all)
        pltpu.sync_copy(dst_hbm.at[pl.ds(wid * nch_w, nch_w)], didx_all)
        plsc.subcore_barrier()

        pltpu.async_copy(y_hbm.at[sidx_all.at[0]], rows0, gsem0)

        def step(i, _):
            c0 = 2 * i
            c1 = c0 + 1
            pltpu.async_copy(y_hbm.at[sidx_all.at[c1]], rows1, gsem1)
            pltpu.make_async_copy(y_hbm.at[sidx_all.at[c0]], rows0, gsem0).wait()
            pltpu.sync_copy(rows0, acc.at[didx_all.at[c0]], add=True)
            cn = jnp.minimum(c0 + 2, nch_w - 1)
            pltpu.async_copy(y_hbm.at[sidx_all.at[cn]], rows0, gsem0)
            pltpu.make_async_copy(y_hbm.at[sidx_all.at[c1]], rows1, gsem1).wait()
            pltpu.sync_copy(rows1, acc.at[didx_all.at[c1]], add=True)
            return ()

        lax.fori_loop(0, npairs, step, ())
        pltpu.make_async_copy(y_hbm.at[sidx_all.at[nch_w - 1]], rows0,
                              gsem0).wait()
        if doubled_idx:
            for parity in range(2):
                lst = list(range(parity, nself, 2))

                @pl.when(cid == parity)
                def _(lst=lst):
                    bufs = ((rows0, gsem0), (rows1, gsem1))
                    for j in range(min(2, len(lst))):
                        pltpu.async_copy(y_hbm.at[sidx_self.at[lst[j]]],
                                         bufs[j][0], bufs[j][1])
                    for j, c in enumerate(lst):
                        buf, sem = bufs[j % 2]
                        pltpu.make_async_copy(y_hbm.at[sidx_self.at[c]],
                                              buf, sem).wait()
                        pltpu.sync_copy(buf, acc.at[didx_self.at[c]],
                                        add=True)
                        if j + 2 < len(lst):
                            pltpu.async_copy(
                                y_hbm.at[sidx_self.at[lst[j + 2]]], buf, sem)
        plsc.subcore_barrier()
        pltpu.sync_copy(acc.at[pl.ds(sid * rows_w, rows_w)],
                        out_hbm.at[cid, pl.ds(sid * rows_w, rows_w)])

    fn = pl.kernel(
        body,
        out_type=jax.ShapeDtypeStruct((_NSC, n_pad, f), jnp.float32),
        mesh=_sc_mesh(),
        compiler_params=pltpu.CompilerParams(use_tc_tiling_on_sc=False),
        scratch_types=[
            pltpu.VMEM((nch_w, _LANES), jnp.int32),
            pltpu.VMEM((nch_w, _LANES), jnp.int32),
            pltpu.VMEM((nself, _LANES), jnp.int32),
            pltpu.VMEM((nself, _LANES), jnp.int32),
            pltpu.VMEM((_LANES, f), jnp.float32),
            pltpu.VMEM((_LANES, f), jnp.float32),
            pltpu.VMEM_SHARED((n_pad, f), jnp.float32),
            pltpu.SemaphoreType.DMA,
            pltpu.SemaphoreType.DMA,
        ],
    )
    return fn(y, src2d, dst2d, zeros_f)


def _tc_scale_first(degv128, x, W, n):
    c_out = W.shape[1]
    xb = 1024
    grid = _cdiv(n, xb)

    def body(degv_ref, x_ref, w_ref, z_ref):
        pid = pl.program_id(0)
        nrow = xb // 128
        deg = (degv_ref[0, pl.ds(nrow * pid, nrow), :]
               + degv_ref[1, pl.ds(nrow * pid, nrow), :]) + 1.0
        dis_t = lax.transpose(lax.rsqrt(deg), (1, 0))
        dcol = jnp.concatenate(
            [dis_t[:, k:k + 1] for k in range(nrow)], axis=0)
        xw = jnp.dot(x_ref[...], w_ref[...],
                     preferred_element_type=jnp.float32)
        z_ref[...] = jnp.concatenate(
            [xw * dcol, jnp.zeros((xb, 128 - c_out), jnp.float32)], axis=1)

    return pl.pallas_call(
        body,
        grid=(grid,),
        in_specs=[
            pl.BlockSpec(degv128.shape, lambda b: (0, 0, 0)),
            pl.BlockSpec((xb, x.shape[1]), lambda b: (b, 0)),
            pl.BlockSpec((x.shape[1], c_out), lambda b: (0, 0)),
        ],
        out_specs=pl.BlockSpec((xb, 128), lambda b: (b, 0)),
        out_shape=jax.ShapeDtypeStruct((n, 128), jnp.float32),
    )(degv128, x, W)


def _tc_mid(up128, degE, n2):
    def body(up_ref, de_ref, v_ref):
        u = up_ref[0, :n2, :] + up_ref[1, :n2, :]
        deg = de_ref[0, :n2, :] + de_ref[1, :n2, :] + 1.0
        v_ref[...] = u / deg

    return pl.pallas_call(
        body,
        out_shape=jax.ShapeDtypeStruct((n2, 128), jnp.float32),
    )(up128, degE)


def _tc_final(wp128, degE, b2, n2):
    c_out = b2.shape[1] // 2

    def body(wp_ref, de_ref, b_ref, o_ref):
        w = wp_ref[0, :n2, :] + wp_ref[1, :n2, :]
        deg = de_ref[0, :n2, :] + de_ref[1, :n2, :] + 1.0
        logits = w * lax.rsqrt(deg) + b_ref[...]

        def lsm(l):
            m = jnp.max(l, axis=-1, keepdims=True)
            ex = jnp.exp(l - m)
            return l - (jnp.log(jnp.sum(ex, axis=-1, keepdims=True)) + m)

        o_ref[...] = jnp.concatenate(
            [lsm(logits[:, :c_out]), lsm(logits[:, c_out:])], axis=1)

    return pl.pallas_call(
        body,
        out_shape=jax.ShapeDtypeStruct((n2, 2 * c_out), jnp.float32),
    )(wp128, degE, b2)


def kernel(x, edge_index, W, b):
    n = x.shape[0]
    c_out = W.shape[1]
    e = edge_index.shape[1]
    n2 = n // 2

    n_pad = _cdiv(n + 1, 8 * _LANES) * 8 * _LANES
    nch = _cdiv(e, _LANES * _NW * 8) * _NW * 8
    nch_w = nch // _NW
    ep = nch * _LANES

    dst_f = _tc_edge_prep_dst(edge_index, n, n_pad, ep)
    src_f, src2_f = _tc_edge_prep_src(edge_index, n, ep)
    src_p = src_f.reshape(nch, _LANES)
    src2_p = src2_f.reshape(nch, _LANES)
    dst_p = dst_f.reshape(nch, _LANES)

    rows_w = n_pad // _NSUB
    zeros16 = jnp.zeros((rows_w, 16), jnp.float32)
    zerosf = jnp.zeros((rows_w, c_out), jnp.float32)
    ones16 = jnp.ones((_LANES, 16), jnp.float32)
    b2 = jnp.concatenate([b, b]).reshape(1, 2 * c_out)

    degv, degE = _sc_degree(dst_p, zeros16, ones16, n_pad, nch_w)
    degv128 = degv.reshape(_NSC, n_pad // _LANES, _LANES)
    zwide = _tc_scale_first(degv128, x, W, n)
    up = _sc_spmm(zwide.reshape(2 * n, c_out), True,
                  src2_p, dst_p, zerosf, n, n_pad, nch_w)
    v128 = _tc_mid(up.reshape(_NSC, n_pad // 2, 2 * c_out), degE, n2)
    wp = _sc_spmm(v128.reshape(n, c_out), False,
                  src_p, dst_p, zerosf, n, n_pad, nch_w)
    out128 = _tc_final(wp.reshape(_NSC, n_pad // 2, 2 * c_out), degE, b2, n2)
    return out128.reshape(n, c_out)


# speedup vs baseline: 1.2847x; 1.0952x over previous
import jax
import jax.numpy as jnp
from jax import lax
from jax.experimental import pallas as pl
from jax.experimental.pallas import tpu as pltpu
from jax.experimental.pallas import tpu_sc as plsc

_LANES = 128
_NSC = 2
_NSUB = 16
_NW = _NSC * _NSUB


def _cdiv(a, b):
    return (a + b - 1) // b


def _sc_mesh():
    return plsc.VectorSubcoreMesh(core_axis_name="c", subcore_axis_name="s")


def _tc_edge_prep_dst(edge_index, n, n_pad, ep):
    e = edge_index.shape[1]
    blk = 65536
    grid = ep // blk
    trash = n_pad - n

    def body(ei_ref, d_ref):
        gi = pl.program_id(0) * blk + lax.broadcasted_iota(jnp.int32, (blk,), 0)
        d_ref[...] = jnp.where(gi < e, ei_ref[1, :], n + gi % trash)

    return pl.pallas_call(
        body,
        grid=(grid,),
        in_specs=[pl.BlockSpec((2, blk), lambda b: (0, b))],
        out_specs=pl.BlockSpec((blk,), lambda b: (b,)),
        out_shape=jax.ShapeDtypeStruct((ep,), jnp.int32),
    )(edge_index)


def _tc_edge_prep_src(edge_index, n, ep):
    e = edge_index.shape[1]
    blk = 65536
    grid = ep // blk

    def body(ei_ref, s_ref, s2_ref):
        gi = pl.program_id(0) * blk + lax.broadcasted_iota(jnp.int32, (blk,), 0)
        s = jnp.where(gi < e, ei_ref[0, :], gi % n)
        s_ref[...] = s
        s2_ref[...] = 2 * s

    return pl.pallas_call(
        body,
        grid=(grid,),
        in_specs=[pl.BlockSpec((2, blk), lambda b: (0, b))],
        out_specs=[pl.BlockSpec((blk,), lambda b: (b,))] * 2,
        out_shape=[jax.ShapeDtypeStruct((ep,), jnp.int32)] * 2,
    )(edge_index)


def _sc_degree(dst1d, zeros16, ones16, n_pad, nch_w):
    rows_w = n_pad // _NSUB
    ngrp = rows_w // 16
    npair_w = rows_w // 2
    SB = 2 * _LANES
    ew = nch_w * _LANES
    nsb = ew // SB

    def body(dst_hbm, zeros_hbm, ones_hbm, outv_hbm, oute_hbm,
             didx_all, ones_v, cnt_v, deg_v, dege_v, acc, ssem):
        cid = lax.axis_index("c")
        sid = lax.axis_index("s")
        wid = cid * _NSUB + sid
        pltpu.sync_copy(zeros_hbm, acc.at[pl.ds(sid * rows_w, rows_w)])
        pltpu.sync_copy(ones_hbm, ones_v)
        pltpu.sync_copy(dst_hbm.at[pl.ds(wid * ew, ew)], didx_all)
        plsc.subcore_barrier()

        def fire(ci, _):
            pltpu.async_copy(ones_v, acc.at[didx_all.at[pl.ds(ci * SB, SB)]],
                             ssem, add=True)
            return ()

        def drain(ci, _):
            pltpu.make_async_copy(
                ones_v, acc.at[didx_all.at[pl.ds(ci * SB, SB)]], ssem).wait()
            return ()

        lax.fori_loop(0, nsb, fire, ())
        lax.fori_loop(0, nsb, drain, ())
        plsc.subcore_barrier()

        pltpu.sync_copy(acc.at[pl.ds(sid * rows_w, rows_w)], cnt_v)
        riota = lax.iota(jnp.int32, 16)
        zidx = jnp.zeros((16,), jnp.int32)

        def compress(g, _):
            vals = plsc.load_gather(cnt_v, [g * 16 + riota, zidx])
            deg_v[pl.ds(g * 16, 16)] = vals
            return ()

        lax.fori_loop(0, ngrp, compress, ())
        pltpu.sync_copy(deg_v, outv_hbm.at[cid, pl.ds(sid * rows_w, rows_w)])

        def expand(r, _):
            v0 = cnt_v[2 * r, :]
            v1 = cnt_v[2 * r + 1, :]
            for k in range(4):
                dege_v[r, pl.ds(16 * k, 16)] = v0
            for k in range(4, 8):
                dege_v[r, pl.ds(16 * k, 16)] = v1
            return ()

        lax.fori_loop(0, npair_w, expand, ())
        pltpu.sync_copy(dege_v, oute_hbm.at[cid, pl.ds(sid * npair_w, npair_w)])

    fn = pl.kernel(
        body,
        out_type=[jax.ShapeDtypeStruct((_NSC, n_pad), jnp.float32),
                  jax.ShapeDtypeStruct((_NSC, n_pad // 2, 128), jnp.float32)],
        mesh=_sc_mesh(),
        compiler_params=pltpu.CompilerParams(use_tc_tiling_on_sc=False,
                                             needs_layout_passes=False),
        scratch_types=[
            pltpu.VMEM((ew,), jnp.int32),
            pltpu.VMEM((SB, 16), jnp.float32),
            pltpu.VMEM((rows_w, 16), jnp.float32),
            pltpu.VMEM((rows_w,), jnp.float32),
            pltpu.VMEM((npair_w, 128), jnp.float32),
            pltpu.VMEM_SHARED((n_pad, 16), jnp.float32),
            pltpu.SemaphoreType.DMA,
        ],
    )
    return fn(dst1d, zeros16, ones16)


def _sc_spmm(y, doubled_idx, src1d, dst1d, zeros_f, n, n_pad, nch_w):
    f = y.shape[1]
    rows_w = n_pad // _NSUB
    SB = 2 * _LANES
    ew = nch_w * _LANES
    nsb = ew // SB
    nself = rows_w // _LANES
    full_tiles = n // rows_w
    rem = n % rows_w

    def body(y_hbm, src_hbm, dst_hbm, zeros_hbm, out_hbm,
             sidx_all, didx_all, sidx_self, didx_self, rows0, rows1, acc,
             gsem0, gsem1):
        cid = lax.axis_index("c")
        sid = lax.axis_index("s")
        wid = cid * _NSUB + sid

        if doubled_idx:
            pltpu.sync_copy(zeros_hbm, acc.at[pl.ds(sid * rows_w, rows_w)])
            riota = lax.iota(jnp.int32, 16)
            base_node = sid * rows_w
            for c in range(nself):
                for g in range(8):
                    nodes = base_node + (c * 128 + g * 16) + riota
                    didx_self[c, pl.ds(16 * g, 16)] = nodes
                    sidx_self[c, pl.ds(16 * g, 16)] = (
                        jnp.minimum(nodes, n - 1) * 2)
        else:
            @pl.when(jnp.logical_and(cid == 0, sid < full_tiles))
            def _():
                pltpu.sync_copy(y_hbm.at[pl.ds(sid * rows_w, rows_w)],
                                acc.at[pl.ds(sid * rows_w, rows_w)])

            @pl.when(jnp.logical_and(cid == 0, sid >= full_tiles))
            def _():
                if rem:
                    pltpu.sync_copy(y_hbm.at[pl.ds(sid * rows_w, rem)],
                                    acc.at[pl.ds(sid * rows_w, rem)])
                pltpu.sync_copy(
                    zeros_hbm.at[pl.ds(0, rows_w - rem)],
                    acc.at[pl.ds(sid * rows_w + rem, rows_w - rem)])

            @pl.when(cid != 0)
            def _():
                pltpu.sync_copy(zeros_hbm,
                                acc.at[pl.ds(sid * rows_w, rows_w)])

        pltpu.sync_copy(src_hbm.at[pl.ds(wid * ew, ew)], sidx_all)
        pltpu.sync_copy(dst_hbm.at[pl.ds(wid * ew, ew)], didx_all)
        plsc.subcore_barrier()

        def sidx(p):
            return sidx_all.at[pl.ds(p * SB, SB)]

        def didx(p):
            return didx_all.at[pl.ds(p * SB, SB)]

        pltpu.async_copy(y_hbm.at[sidx(0)], rows0, gsem0)

        def step(i, _):
            p0 = 2 * i
            p1 = p0 + 1
            pltpu.async_copy(y_hbm.at[sidx(p1)], rows1, gsem1)
            pltpu.make_async_copy(y_hbm.at[sidx(p0)], rows0, gsem0).wait()
            pltpu.sync_copy(rows0, acc.at[didx(p0)], add=True)
            pn = jnp.minimum(p0 + 2, nsb - 1)
            pltpu.async_copy(y_hbm.at[sidx(pn)], rows0, gsem0)
            pltpu.make_async_copy(y_hbm.at[sidx(p1)], rows1, gsem1).wait()
            pltpu.sync_copy(rows1, acc.at[didx(p1)], add=True)
            return ()

        lax.fori_loop(0, nsb // 2, step, ())
        pltpu.make_async_copy(y_hbm.at[sidx(nsb - 1)], rows0, gsem0).wait()
        if doubled_idx:
            for parity in range(2):
                lst = list(range(parity, nself, 2))

                @pl.when(cid == parity)
                def _(lst=lst):
                    bufs = ((rows0.at[pl.ds(0, _LANES)], gsem0),
                            (rows1.at[pl.ds(0, _LANES)], gsem1))
                    for j in range(min(2, len(lst))):
                        pltpu.async_copy(y_hbm.at[sidx_self.at[lst[j]]],
                                         bufs[j][0], bufs[j][1])
                    for j, c in enumerate(lst):
                        buf, sem = bufs[j % 2]
                        pltpu.make_async_copy(y_hbm.at[sidx_self.at[c]],
                                              buf, sem).wait()
                        pltpu.sync_copy(buf, acc.at[didx_self.at[c]],
                                        add=True)
                        if j + 2 < len(lst):
                            pltpu.async_copy(
                                y_hbm.at[sidx_self.at[lst[j + 2]]], buf, sem)
        plsc.subcore_barrier()
        pltpu.sync_copy(acc.at[pl.ds(sid * rows_w, rows_w)],
                        out_hbm.at[cid, pl.ds(sid * rows_w, rows_w)])

    fn = pl.kernel(
        body,
        out_type=jax.ShapeDtypeStruct((_NSC, n_pad, f), jnp.float32),
        mesh=_sc_mesh(),
        compiler_params=pltpu.CompilerParams(use_tc_tiling_on_sc=False),
        scratch_types=[
            pltpu.VMEM((ew,), jnp.int32),
            pltpu.VMEM((ew,), jnp.int32),
            pltpu.VMEM((nself, _LANES), jnp.int32),
            pltpu.VMEM((nself, _LANES), jnp.int32),
            pltpu.VMEM((SB, f), jnp.float32),
            pltpu.VMEM((SB, f), jnp.float32),
            pltpu.VMEM_SHARED((n_pad, f), jnp.float32),
            pltpu.SemaphoreType.DMA,
            pltpu.SemaphoreType.DMA,
        ],
    )
    return fn(y, src1d, dst1d, zeros_f)


def _tc_scale_first(degv128, x, W, n):
    c_out = W.shape[1]
    xb = 1024
    grid = _cdiv(n, xb)

    def body(degv_ref, x_ref, w_ref, z_ref):
        pid = pl.program_id(0)
        nrow = xb // 128
        deg = (degv_ref[0, pl.ds(nrow * pid, nrow), :]
               + degv_ref[1, pl.ds(nrow * pid, nrow), :]) + 1.0
        dis_t = lax.transpose(lax.rsqrt(deg), (1, 0))
        dcol = jnp.concatenate(
            [dis_t[:, k:k + 1] for k in range(nrow)], axis=0)
        xw = jnp.dot(x_ref[...], w_ref[...],
                     preferred_element_type=jnp.float32)
        z_ref[...] = jnp.concatenate(
            [xw * dcol, jnp.zeros((xb, 128 - c_out), jnp.float32)], axis=1)

    return pl.pallas_call(
        body,
        grid=(grid,),
        in_specs=[
            pl.BlockSpec(degv128.shape, lambda b: (0, 0, 0)),
            pl.BlockSpec((xb, x.shape[1]), lambda b: (b, 0)),
            pl.BlockSpec((x.shape[1], c_out), lambda b: (0, 0)),
        ],
        out_specs=pl.BlockSpec((xb, 128), lambda b: (b, 0)),
        out_shape=jax.ShapeDtypeStruct((n, 128), jnp.float32),
    )(degv128, x, W)


def _tc_mid(up128, degE, n2):
    def body(up_ref, de_ref, v_ref):
        u = up_ref[0, :n2, :] + up_ref[1, :n2, :]
        deg = de_ref[0, :n2, :] + de_ref[1, :n2, :] + 1.0
        v_ref[...] = u / deg

    return pl.pallas_call(
        body,
        out_shape=jax.ShapeDtypeStruct((n2, 128), jnp.float32),
    )(up128, degE)


def _tc_final(wp128, degE, b2, n2):
    c_out = b2.shape[1] // 2

    def body(wp_ref, de_ref, b_ref, o_ref):
        w = wp_ref[0, :n2, :] + wp_ref[1, :n2, :]
        deg = de_ref[0, :n2, :] + de_ref[1, :n2, :] + 1.0
        logits = w * lax.rsqrt(deg) + b_ref[...]

        def lsm(l):
            m = jnp.max(l, axis=-1, keepdims=True)
            ex = jnp.exp(l - m)
            return l - (jnp.log(jnp.sum(ex, axis=-1, keepdims=True)) + m)

        o_ref[...] = jnp.concatenate(
            [lsm(logits[:, :c_out]), lsm(logits[:, c_out:])], axis=1)

    return pl.pallas_call(
        body,
        out_shape=jax.ShapeDtypeStruct((n2, 2 * c_out), jnp.float32),
    )(wp128, degE, b2)


def kernel(x, edge_index, W, b):
    n = x.shape[0]
    c_out = W.shape[1]
    e = edge_index.shape[1]
    n2 = n // 2

    n_pad = _cdiv(n + 1, 8 * _LANES) * 8 * _LANES
    nch = _cdiv(e, _LANES * _NW * 8) * _NW * 8
    nch_w = nch // _NW
    ep = nch * _LANES

    dst_f = _tc_edge_prep_dst(edge_index, n, n_pad, ep)
    src_f, src2_f = _tc_edge_prep_src(edge_index, n, ep)

    rows_w = n_pad // _NSUB
    zeros16 = jnp.zeros((rows_w, 16), jnp.float32)
    zerosf = jnp.zeros((rows_w, c_out), jnp.float32)
    ones16 = jnp.ones((2 * _LANES, 16), jnp.float32)
    b2 = jnp.concatenate([b, b]).reshape(1, 2 * c_out)

    degv, degE = _sc_degree(dst_f, zeros16, ones16, n_pad, nch_w)
    degv128 = degv.reshape(_NSC, n_pad // _LANES, _LANES)
    zwide = _tc_scale_first(degv128, x, W, n)
    up = _sc_spmm(zwide.reshape(2 * n, c_out), True,
                  src2_f, dst_f, zerosf, n, n_pad, nch_w)
    v128 = _tc_mid(up.reshape(_NSC, n_pad // 2, 2 * c_out), degE, n2)
    wp = _sc_spmm(v128.reshape(n, c_out), False,
                  src_f, dst_f, zerosf, n, n_pad, nch_w)
    out128 = _tc_final(wp.reshape(_NSC, n_pad // 2, 2 * c_out), degE, b2, n2)
    return out128.reshape(n, c_out)
